# baseline (speedup 1.0000x reference)
import functools

import jax
import jax.numpy as jnp
from jax import lax
from jax.experimental import pallas as pl
from jax.experimental.pallas import tpu as pltpu

_N = 4096
_E = 4
_DIN = 128
_DOUT = 64
_NC = 8
_B = 1024

_BB = 128
_NB = _B // _BB
_BM = 512


def _prep_body(relw_ref, x_ref, wsa_ref, w1_ref, u1_ref, vp_ref, ws_ref):
    x = x_ref[:]
    u1_ref[:] = jnp.dot(x, w1_ref[:], preferred_element_type=jnp.float32)
    vsa = jnp.dot(x, wsa_ref[:], preferred_element_type=jnp.float32)
    rw = relw_ref[:]
    m = jnp.max(rw)
    ew = jnp.exp(rw - m)
    ws = ew / jnp.sum(ew)
    ws_ref[:] = ws
    for e in range(_E):
        vp_ref[:, e, :] = vsa * ws[0:1, e:e + 1]


def _zv1_body(h_ref, u1_ref, o_ref):
    o_ref[:] = jnp.maximum(
        jnp.dot(h_ref[:], u1_ref[:], preferred_element_type=jnp.float32), 0.0)


def _main_body(tx_ref, a_ref, h_ref, zv1_ref, w2_ref, vp_ref, l1w_ref,
               l1b_ref, l2w_ref, l2b_ref, tgt_ref, y_ref, loss_ref,
               u2_ref, abuf, hbuf, asem, hsem, lacc):
    b = pl.program_id(0)
    nb = pl.num_programs(0)

    def issue(slot, tile):
        def ibody(i, _):
            r = tx_ref[tile * _BB + i]
            pltpu.make_async_copy(a_ref.at[r], abuf.at[slot, i],
                                  asem.at[slot]).start()
            pltpu.make_async_copy(h_ref.at[r], hbuf.at[slot, i],
                                  hsem.at[slot]).start()
            return 0
        lax.fori_loop(0, _BB, ibody, 0)

    @pl.when(b == 0)
    def _():
        u2_ref[:] = jnp.dot(zv1_ref[:], w2_ref[:],
                            preferred_element_type=jnp.float32)
        issue(0, 0)

    @pl.when(b + 1 < nb)
    def _():
        issue((b + 1) % 2, b + 1)

    slot = b % 2
    pltpu.make_async_copy(abuf.at[slot], abuf.at[slot], asem.at[slot]).wait()
    pltpu.make_async_copy(hbuf.at[slot], hbuf.at[slot], hsem.at[slot]).wait()

    ag = abuf[slot]
    hg = hbuf[slot]
    zr = jnp.maximum(
        jnp.dot(ag, vp_ref[:], preferred_element_type=jnp.float32), 0.0)
    zv = jnp.dot(hg, u2_ref[:], preferred_element_type=jnp.float32)
    nrm = jnp.sqrt(jnp.sum(zr * zr, axis=1, keepdims=True) +
                   jnp.sum(zv * zv, axis=1, keepdims=True))
    inv = 1.0 / jnp.maximum(nrm, 1e-12)
    zrn = zr * inv
    zvn = zv * inv
    z1 = jnp.maximum(
        jnp.dot(zrn, l1w_ref[0:_DOUT, :], preferred_element_type=jnp.float32)
        + jnp.dot(zvn, l1w_ref[_DOUT:, :], preferred_element_type=jnp.float32)
        + l1b_ref[:], 0.0)
    yb = jnp.dot(z1, l2w_ref[:], preferred_element_type=jnp.float32) \
        + l2b_ref[:]
    y_ref[:] = yb
    m = jnp.max(yb, axis=1, keepdims=True)
    lse = m + jnp.log(jnp.sum(jnp.exp(yb - m), axis=1, keepdims=True))
    logp = yb - lse
    t = tgt_ref[0, 0, :]
    sel = t[:, None] == lax.broadcasted_iota(jnp.int32, (_BB, _NC), 1)
    contrib = jnp.sum(jnp.where(sel, logp, 0.0))
    prev = jnp.where(b == 0, 0.0, lacc[0])
    lacc[0] = prev + contrib

    @pl.when(b == nb - 1)
    def _():
        loss_ref[0, 0] = -lacc[0] / _B


def kernel(A, H_adj, X, target_x, target, rel_weight, W_sa, W1, W2,
           lin1_w, lin1_b, lin2_w, lin2_b):
    f32 = jnp.float32
    relw = rel_weight.reshape(1, _E)

    u1, vp, ws = pl.pallas_call(
        _prep_body,
        out_shape=[
            jax.ShapeDtypeStruct((_N, _DOUT), f32),
            jax.ShapeDtypeStruct((_N, _E, _DOUT), f32),
            jax.ShapeDtypeStruct((1, _E), f32),
        ],
    )(relw, X, W_sa, W1)

    zv1 = pl.pallas_call(
        _zv1_body,
        grid=(_N // _BM,),
        in_specs=[
            pl.BlockSpec((_BM, _N), lambda i: (i, 0)),
            pl.BlockSpec((_N, _DOUT), lambda i: (0, 0)),
        ],
        out_specs=pl.BlockSpec((_BM, _DOUT), lambda i: (i, 0)),
        out_shape=jax.ShapeDtypeStruct((_N, _DOUT), f32),
    )(H_adj, u1)

    a2d = A.reshape(_N, _N * _E)
    vprime = vp.reshape(_N * _E, _DOUT)
    tgt3d = target.astype(jnp.int32).reshape(_NB, 1, _BB)
    tx = target_x.astype(jnp.int32)

    y, loss = pl.pallas_call(
        _main_body,
        grid_spec=pltpu.PrefetchScalarGridSpec(
            num_scalar_prefetch=1,
            grid=(_NB,),
            in_specs=[
                pl.BlockSpec(memory_space=pl.ANY),
                pl.BlockSpec(memory_space=pl.ANY),
                pl.BlockSpec((_N, _DOUT), lambda b, tx: (0, 0)),
                pl.BlockSpec((_DOUT, _DOUT), lambda b, tx: (0, 0)),
                pl.BlockSpec((_N * _E, _DOUT), lambda b, tx: (0, 0)),
                pl.BlockSpec((2 * _DOUT, _DOUT), lambda b, tx: (0, 0)),
                pl.BlockSpec((1, _DOUT), lambda b, tx: (0, 0)),
                pl.BlockSpec((_DOUT, _NC), lambda b, tx: (0, 0)),
                pl.BlockSpec((1, _NC), lambda b, tx: (0, 0)),
                pl.BlockSpec((1, 1, _BB), lambda b, tx: (b, 0, 0)),
            ],
            out_specs=[
                pl.BlockSpec((_BB, _NC), lambda b, tx: (b, 0)),
                pl.BlockSpec((1, 1), lambda b, tx: (0, 0),
                             memory_space=pltpu.SMEM),
            ],
            scratch_shapes=[
                pltpu.VMEM((_N, _DOUT), f32),
                pltpu.VMEM((2, _BB, _N * _E), f32),
                pltpu.VMEM((2, _BB, _N), f32),
                pltpu.SemaphoreType.DMA((2,)),
                pltpu.SemaphoreType.DMA((2,)),
                pltpu.SMEM((1,), f32),
            ],
        ),
        out_shape=[
            jax.ShapeDtypeStruct((_B, _NC), f32),
            jax.ShapeDtypeStruct((1, 1), f32),
        ],
        compiler_params=pltpu.CompilerParams(
            dimension_semantics=("arbitrary",)),
    )(tx, a2d, H_adj, zv1, W2, vprime, lin1_w, lin1_b.reshape(1, _DOUT),
      lin2_w, lin2_b.reshape(1, _NC), tgt3d)

    return (loss[0, 0], y, ws.reshape(1, _E, 1, 1))


# speedup vs baseline: 10.0198x; 10.0198x over previous
import functools

import jax
import jax.numpy as jnp
from jax import lax
from jax.experimental import pallas as pl
from jax.experimental.pallas import tpu as pltpu

_N = 4096
_E = 4
_DIN = 128
_DOUT = 64
_NC = 8
_B = 1024

_BB = 128
_NB = _B // _BB
_BM = 512


def _prep_body(relw_ref, x_ref, wsa_ref, w1_ref, u1_ref, vp_ref, ws_ref):
    x = x_ref[:]
    u1_ref[:] = jnp.dot(x, w1_ref[:], preferred_element_type=jnp.float32)
    vsa = jnp.dot(x, wsa_ref[:], preferred_element_type=jnp.float32)
    rw = relw_ref[:]
    m = jnp.max(rw)
    ew = jnp.exp(rw - m)
    ws = ew / jnp.sum(ew)
    ws_ref[:] = ws
    vsa4 = vsa.reshape(_N // 128, 128, _DOUT)
    for e in range(_E):
        vp_ref[:, e, :, :] = vsa4 * ws[0:1, e:e + 1, None]


def _zv1_body(h_ref, u1_ref, o_ref):
    o_ref[:] = jnp.maximum(
        jnp.dot(h_ref[:], u1_ref[:], preferred_element_type=jnp.float32), 0.0)


def _main_body(tx_ref, a_ref, h_ref, zv1_ref, w2_ref, vp_ref, l1w_ref,
               l1b_ref, l2w_ref, l2b_ref, tgt_ref, y_ref, loss_ref,
               u2_ref, abuf, hbuf, asem, hsem, lacc):
    b = pl.program_id(0)
    nb = pl.num_programs(0)

    def issue(slot, tile):
        def ibody(i, _):
            r = tx_ref[tile * _BB + i]
            pltpu.make_async_copy(a_ref.at[pl.ds(r * (_N * _E), _N * _E)],
                                  abuf.at[slot, i],
                                  asem.at[slot]).start()
            pltpu.make_async_copy(h_ref.at[r], hbuf.at[slot, i],
                                  hsem.at[slot]).start()
            return 0
        lax.fori_loop(0, _BB, ibody, 0)

    @pl.when(b == 0)
    def _():
        u2_ref[:] = jnp.dot(zv1_ref[:], w2_ref[:],
                            preferred_element_type=jnp.float32)
        issue(0, 0)

    @pl.when(b + 1 < nb)
    def _():
        issue((b + 1) % 2, b + 1)

    slot = b % 2
    pltpu.make_async_copy(abuf.at[slot], abuf.at[slot], asem.at[slot]).wait()
    pltpu.make_async_copy(hbuf.at[slot], hbuf.at[slot], hsem.at[slot]).wait()

    ag = abuf[slot]
    hg = hbuf[slot]
    zr = jnp.maximum(
        jnp.dot(ag, vp_ref[:], preferred_element_type=jnp.float32), 0.0)
    zv = jnp.dot(hg, u2_ref[:], preferred_element_type=jnp.float32)
    nrm = jnp.sqrt(jnp.sum(zr * zr, axis=1, keepdims=True) +
                   jnp.sum(zv * zv, axis=1, keepdims=True))
    inv = 1.0 / jnp.maximum(nrm, 1e-12)
    zrn = zr * inv
    zvn = zv * inv
    z1 = jnp.maximum(
        jnp.dot(zrn, l1w_ref[0:_DOUT, :], preferred_element_type=jnp.float32)
        + jnp.dot(zvn, l1w_ref[_DOUT:, :], preferred_element_type=jnp.float32)
        + l1b_ref[:], 0.0)
    yb = jnp.dot(z1, l2w_ref[:], preferred_element_type=jnp.float32) \
        + l2b_ref[:]
    y_ref[:] = yb
    m = jnp.max(yb, axis=1, keepdims=True)
    lse = m + jnp.log(jnp.sum(jnp.exp(yb - m), axis=1, keepdims=True))
    logp = yb - lse
    t = tgt_ref[0, 0, :]
    sel = t[:, None] == lax.broadcasted_iota(jnp.int32, (_BB, _NC), 1)
    contrib = jnp.sum(jnp.where(sel, logp, 0.0))
    prev = jnp.where(b == 0, 0.0, lacc[0])
    lacc[0] = prev + contrib

    @pl.when(b == nb - 1)
    def _():
        loss_ref[0, 0] = -lacc[0] / _B


def kernel(A, H_adj, X, target_x, target, rel_weight, W_sa, W1, W2,
           lin1_w, lin1_b, lin2_w, lin2_b):
    f32 = jnp.float32
    relw = rel_weight.reshape(1, _E)

    u1, vp, ws = pl.pallas_call(
        _prep_body,
        out_shape=[
            jax.ShapeDtypeStruct((_N, _DOUT), f32),
            jax.ShapeDtypeStruct((_N // 128, _E, 128, _DOUT), f32),
            jax.ShapeDtypeStruct((1, _E), f32),
        ],
    )(relw, X, W_sa, W1)

    zv1 = pl.pallas_call(
        _zv1_body,
        grid=(_N // _BM,),
        in_specs=[
            pl.BlockSpec((_BM, _N), lambda i: (i, 0)),
            pl.BlockSpec((_N, _DOUT), lambda i: (0, 0)),
        ],
        out_specs=pl.BlockSpec((_BM, _DOUT), lambda i: (i, 0)),
        out_shape=jax.ShapeDtypeStruct((_N, _DOUT), f32),
    )(H_adj, u1)

    a_lin = A.reshape(_N, _N // 128, 128, _E).transpose(0, 1, 3, 2) \
        .reshape(_N * _N * _E)
    vprime = vp.reshape(_N * _E, _DOUT)
    tgt3d = target.astype(jnp.int32).reshape(_NB, 1, _BB)
    tx = target_x.astype(jnp.int32)

    y, loss = pl.pallas_call(
        _main_body,
        grid_spec=pltpu.PrefetchScalarGridSpec(
            num_scalar_prefetch=1,
            grid=(_NB,),
            in_specs=[
                pl.BlockSpec(memory_space=pl.ANY),
                pl.BlockSpec(memory_space=pl.ANY),
                pl.BlockSpec((_N, _DOUT), lambda b, tx: (0, 0)),
                pl.BlockSpec((_DOUT, _DOUT), lambda b, tx: (0, 0)),
                pl.BlockSpec((_N * _E, _DOUT), lambda b, tx: (0, 0)),
                pl.BlockSpec((2 * _DOUT, _DOUT), lambda b, tx: (0, 0)),
                pl.BlockSpec((1, _DOUT), lambda b, tx: (0, 0)),
                pl.BlockSpec((_DOUT, _NC), lambda b, tx: (0, 0)),
                pl.BlockSpec((1, _NC), lambda b, tx: (0, 0)),
                pl.BlockSpec((1, 1, _BB), lambda b, tx: (b, 0, 0)),
            ],
            out_specs=[
                pl.BlockSpec((_BB, _NC), lambda b, tx: (b, 0)),
                pl.BlockSpec((1, 1), lambda b, tx: (0, 0),
                             memory_space=pltpu.SMEM),
            ],
            scratch_shapes=[
                pltpu.VMEM((_N, _DOUT), f32),
                pltpu.VMEM((2, _BB, _N * _E), f32),
                pltpu.VMEM((2, _BB, _N), f32),
                pltpu.SemaphoreType.DMA((2,)),
                pltpu.SemaphoreType.DMA((2,)),
                pltpu.SMEM((1,), f32),
            ],
        ),
        out_shape=[
            jax.ShapeDtypeStruct((_B, _NC), f32),
            jax.ShapeDtypeStruct((1, 1), f32),
        ],
        compiler_params=pltpu.CompilerParams(
            dimension_semantics=("arbitrary",)),
    )(tx, a_lin, H_adj, zv1, W2, vprime, lin1_w, lin1_b.reshape(1, _DOUT),
      lin2_w, lin2_b.reshape(1, _NC), tgt3d)

    return (loss[0, 0], y, ws.reshape(1, _E, 1, 1))


# speedup vs baseline: 10.0983x; 1.0078x over previous
import jax
import jax.numpy as jnp
from jax import lax
from jax.experimental import pallas as pl
from jax.experimental.pallas import tpu as pltpu

_N = 4096
_E = 4
_DIN = 128
_DOUT = 64
_NC = 8
_B = 1024

_BB = 128
_NT = _B // _BB
_BMZ = 256
_NZ = _N // _BMZ


def _prep_body(relw_ref, x_ref, wsa_ref, w1_ref, u1_ref, vp_ref, ws_ref):
    x = x_ref[:]
    u1_ref[:] = jnp.dot(x, w1_ref[:], preferred_element_type=jnp.float32)
    vsa = jnp.dot(x, wsa_ref[:], preferred_element_type=jnp.float32)
    rw = relw_ref[:]
    m = jnp.max(rw)
    ew = jnp.exp(rw - m)
    ws = ew / jnp.sum(ew)
    ws_ref[:] = ws
    vsa4 = vsa.reshape(_N // 128, 128, _DOUT)
    for e in range(_E):
        vp_ref[:, e, :, :] = vsa4 * ws[0:1, e:e + 1, None]


def _fused_body(tx_ref, a_ref, h_ref, u1_ref, w2_ref, vp_ref, l1w_ref,
                l1b_ref, l2w_ref, l2b_ref, tgt_ref, y_ref, loss_ref,
                zv1_ref, u2_ref, hslab, abuf, hbuf, zsem, asem, hsem, lacc):
    g = pl.program_id(0)

    def issue_tile(slot, tile):
        def ibody(i, _):
            r = tx_ref[tile * _BB + i]
            pltpu.make_async_copy(a_ref.at[pl.ds(r * (_N * _E), _N * _E)],
                                  abuf.at[slot, i],
                                  asem.at[slot]).start()
            pltpu.make_async_copy(h_ref.at[r], hbuf.at[slot, i],
                                  hsem.at[slot]).start()
            return 0
        lax.fori_loop(0, _BB, ibody, 0)

    def issue_slab(slot, s):
        pltpu.make_async_copy(h_ref.at[pl.ds(s * _BMZ, _BMZ)],
                              hslab.at[slot], zsem.at[slot]).start()

    @pl.when(g == 0)
    def _():
        issue_slab(0, 0)
        issue_slab(1, 1)

    @pl.when(g == _NZ - 2)
    def _():
        issue_tile(0, 0)

    @pl.when(g < _NZ)
    def _():
        slot = g % 2
        pltpu.make_async_copy(hslab.at[slot], hslab.at[slot],
                              zsem.at[slot]).wait()
        zv1_ref[pl.ds(g * _BMZ, _BMZ), :] = jnp.maximum(
            jnp.dot(hslab[slot], u1_ref[:],
                    preferred_element_type=jnp.float32), 0.0)

        @pl.when(g + 2 < _NZ)
        def _():
            issue_slab(slot, g + 2)

    @pl.when(g >= _NZ)
    def _():
        t = g - _NZ

        @pl.when(t == 0)
        def _():
            u2_ref[:] = jnp.dot(zv1_ref[:], w2_ref[:],
                                preferred_element_type=jnp.float32)

        @pl.when(t + 1 < _NT)
        def _():
            issue_tile((t + 1) % 2, t + 1)

        slot = t % 2
        pltpu.make_async_copy(abuf.at[slot], abuf.at[slot],
                              asem.at[slot]).wait()
        pltpu.make_async_copy(hbuf.at[slot], hbuf.at[slot],
                              hsem.at[slot]).wait()

        ag = abuf[slot]
        hg = hbuf[slot]
        zr = jnp.maximum(
            jnp.dot(ag, vp_ref[:], preferred_element_type=jnp.float32), 0.0)
        zv = jnp.dot(hg, u2_ref[:], preferred_element_type=jnp.float32)
        nrm = jnp.sqrt(jnp.sum(zr * zr, axis=1, keepdims=True) +
                       jnp.sum(zv * zv, axis=1, keepdims=True))
        inv = 1.0 / jnp.maximum(nrm, 1e-12)
        zrn = zr * inv
        zvn = zv * inv
        z1 = jnp.maximum(
            jnp.dot(zrn, l1w_ref[0:_DOUT, :],
                    preferred_element_type=jnp.float32)
            + jnp.dot(zvn, l1w_ref[_DOUT:, :],
                      preferred_element_type=jnp.float32)
            + l1b_ref[:], 0.0)
        yb = jnp.dot(z1, l2w_ref[:], preferred_element_type=jnp.float32) \
            + l2b_ref[:]
        y_ref[:] = yb
        m = jnp.max(yb, axis=1, keepdims=True)
        lse = m + jnp.log(jnp.sum(jnp.exp(yb - m), axis=1, keepdims=True))
        logp = yb - lse
        tt = tgt_ref[0, 0, :]
        sel = tt[:, None] == lax.broadcasted_iota(jnp.int32, (_BB, _NC), 1)
        contrib = jnp.sum(jnp.where(sel, logp, 0.0))
        prev = jnp.where(t == 0, 0.0, lacc[0])
        lacc[0] = prev + contrib

        @pl.when(t == _NT - 1)
        def _():
            loss_ref[0, 0] = -lacc[0] / _B


def kernel(A, H_adj, X, target_x, target, rel_weight, W_sa, W1, W2,
           lin1_w, lin1_b, lin2_w, lin2_b):
    f32 = jnp.float32
    relw = rel_weight.reshape(1, _E)

    u1, vp, ws = pl.pallas_call(
        _prep_body,
        out_shape=[
            jax.ShapeDtypeStruct((_N, _DOUT), f32),
            jax.ShapeDtypeStruct((_N // 128, _E, 128, _DOUT), f32),
            jax.ShapeDtypeStruct((1, _E), f32),
        ],
    )(relw, X, W_sa, W1)

    a_lin = A.reshape(_N, _N // 128, 128, _E).transpose(0, 1, 3, 2) \
        .reshape(_N * _N * _E)
    vprime = vp.reshape(_N * _E, _DOUT)
    tgt3d = target.astype(jnp.int32).reshape(_NT, 1, _BB)
    tx = target_x.astype(jnp.int32)

    nsteps = _NZ + _NT

    def _main_map(g, tx):
        del tx
        return (jnp.maximum(g - _NZ, 0),)

    y, loss = pl.pallas_call(
        _fused_body,
        grid_spec=pltpu.PrefetchScalarGridSpec(
            num_scalar_prefetch=1,
            grid=(nsteps,),
            in_specs=[
                pl.BlockSpec(memory_space=pl.ANY),
                pl.BlockSpec(memory_space=pl.ANY),
                pl.BlockSpec((_N, _DOUT), lambda g, tx: (0, 0)),
                pl.BlockSpec((_DOUT, _DOUT), lambda g, tx: (0, 0)),
                pl.BlockSpec((_N * _E, _DOUT), lambda g, tx: (0, 0)),
                pl.BlockSpec((2 * _DOUT, _DOUT), lambda g, tx: (0, 0)),
                pl.BlockSpec((1, _DOUT), lambda g, tx: (0, 0)),
                pl.BlockSpec((_DOUT, _NC), lambda g, tx: (0, 0)),
                pl.BlockSpec((1, _NC), lambda g, tx: (0, 0)),
                pl.BlockSpec((1, 1, _BB),
                             lambda g, tx: (jnp.maximum(g - _NZ, 0), 0, 0)),
            ],
            out_specs=[
                pl.BlockSpec((_BB, _NC),
                             lambda g, tx: (jnp.maximum(g - _NZ, 0), 0)),
                pl.BlockSpec((1, 1), lambda g, tx: (0, 0),
                             memory_space=pltpu.SMEM),
            ],
            scratch_shapes=[
                pltpu.VMEM((_N, _DOUT), f32),
                pltpu.VMEM((_N, _DOUT), f32),
                pltpu.VMEM((2, _BMZ, _N), f32),
                pltpu.VMEM((2, _BB, _N * _E), f32),
                pltpu.VMEM((2, _BB, _N), f32),
                pltpu.SemaphoreType.DMA((2,)),
                pltpu.SemaphoreType.DMA((2,)),
                pltpu.SemaphoreType.DMA((2,)),
                pltpu.SMEM((1,), f32),
            ],
        ),
        out_shape=[
            jax.ShapeDtypeStruct((_B, _NC), f32),
            jax.ShapeDtypeStruct((1, 1), f32),
        ],
        compiler_params=pltpu.CompilerParams(
            dimension_semantics=("arbitrary",)),
    )(tx, a_lin, H_adj, u1, W2, vprime, lin1_w, lin1_b.reshape(1, _DOUT),
      lin2_w, lin2_b.reshape(1, _NC), tgt3d)

    return (loss[0, 0], y, ws.reshape(1, _E, 1, 1))


# speedup vs baseline: 11.2855x; 1.1176x over previous
import jax
import jax.numpy as jnp
from jax import lax
from jax.experimental import pallas as pl
from jax.experimental.pallas import tpu as pltpu

_N = 4096
_E = 4
_DIN = 128
_DOUT = 64
_NC = 8
_B = 1024

_BB = 128
_NT = _B // _BB
_BMZ = 256
_NZ = _N // _BMZ


def _fused_body(tx_ref, a_ref, h_ref, relw_ref, x_ref, wsa_ref, w1_ref,
                w2_ref, l1w_ref, l1b_ref, l2w_ref, l2b_ref, tgt_ref,
                y_ref, loss_ref, ws_ref,
                u1_ref, vp_ref, zv1_ref, u2_ref, hslab, abuf, hbuf,
                zsem, asem, hsem, lacc):
    g = pl.program_id(0)

    def issue_tile(slot, tile):
        def ibody(i, _):
            r = tx_ref[tile * _BB + i]
            pltpu.make_async_copy(a_ref.at[pl.ds(r * (_N * _E), _N * _E)],
                                  abuf.at[slot, i],
                                  asem.at[slot]).start()
            pltpu.make_async_copy(h_ref.at[r], hbuf.at[slot, i],
                                  hsem.at[slot]).start()
            return 0
        lax.fori_loop(0, _BB, ibody, 0)

    def issue_slab(slot, s):
        pltpu.make_async_copy(h_ref.at[pl.ds(s * _BMZ, _BMZ)],
                              hslab.at[slot], zsem.at[slot]).start()

    @pl.when(g == 0)
    def _():
        issue_slab(0, 0)
        issue_slab(1, 1)
        x = x_ref[:]
        u1_ref[:] = jnp.dot(x, w1_ref[:], preferred_element_type=jnp.float32)
        vsa = jnp.dot(x, wsa_ref[:], preferred_element_type=jnp.float32)
        rw = relw_ref[:]
        m = jnp.max(rw)
        ew = jnp.exp(rw - m)
        ws = ew / jnp.sum(ew)
        ws_ref[:] = ws
        for e in range(_E):
            sv = vsa * ws[0:1, e:e + 1]
            for jb in range(_N // 128):
                vp_ref[pl.ds(jb * 512 + e * 128, 128), :] = \
                    sv[jb * 128:(jb + 1) * 128, :]

    @pl.when(g == _NZ - 2)
    def _():
        issue_tile(0, 0)

    @pl.when(g < _NZ)
    def _():
        slot = g % 2
        pltpu.make_async_copy(hslab.at[slot], hslab.at[slot],
                              zsem.at[slot]).wait()
        zv1_ref[pl.ds(g * _BMZ, _BMZ), :] = jnp.maximum(
            jnp.dot(hslab[slot], u1_ref[:],
                    preferred_element_type=jnp.float32), 0.0)

        @pl.when(g + 2 < _NZ)
        def _():
            issue_slab(slot, g + 2)

    @pl.when(g >= _NZ)
    def _():
        t = g - _NZ

        @pl.when(t == 0)
        def _():
            u2_ref[:] = jnp.dot(zv1_ref[:], w2_ref[:],
                                preferred_element_type=jnp.float32)

        @pl.when(t + 1 < _NT)
        def _():
            issue_tile((t + 1) % 2, t + 1)

        slot = t % 2
        pltpu.make_async_copy(abuf.at[slot], abuf.at[slot],
                              asem.at[slot]).wait()
        pltpu.make_async_copy(hbuf.at[slot], hbuf.at[slot],
                              hsem.at[slot]).wait()

        ag = abuf[slot]
        hg = hbuf[slot]
        zr = jnp.maximum(
            jnp.dot(ag, vp_ref[:], preferred_element_type=jnp.float32), 0.0)
        zv = jnp.dot(hg, u2_ref[:], preferred_element_type=jnp.float32)
        nrm = jnp.sqrt(jnp.sum(zr * zr, axis=1, keepdims=True) +
                       jnp.sum(zv * zv, axis=1, keepdims=True))
        inv = 1.0 / jnp.maximum(nrm, 1e-12)
        zrn = zr * inv
        zvn = zv * inv
        z1 = jnp.maximum(
            jnp.dot(zrn, l1w_ref[0:_DOUT, :],
                    preferred_element_type=jnp.float32)
            + jnp.dot(zvn, l1w_ref[_DOUT:, :],
                      preferred_element_type=jnp.float32)
            + l1b_ref[:], 0.0)
        yb = jnp.dot(z1, l2w_ref[:], preferred_element_type=jnp.float32) \
            + l2b_ref[:]
        y_ref[:] = yb
        m = jnp.max(yb, axis=1, keepdims=True)
        lse = m + jnp.log(jnp.sum(jnp.exp(yb - m), axis=1, keepdims=True))
        logp = yb - lse
        tt = tgt_ref[0, 0, :]
        sel = tt[:, None] == lax.broadcasted_iota(jnp.int32, (_BB, _NC), 1)
        contrib = jnp.sum(jnp.where(sel, logp, 0.0))
        prev = jnp.where(t == 0, 0.0, lacc[0])
        lacc[0] = prev + contrib

        @pl.when(t == _NT - 1)
        def _():
            loss_ref[0, 0] = -lacc[0] / _B


def kernel(A, H_adj, X, target_x, target, rel_weight, W_sa, W1, W2,
           lin1_w, lin1_b, lin2_w, lin2_b):
    f32 = jnp.float32
    relw = rel_weight.reshape(1, _E)

    a_lin = A.reshape(_N, _N // 128, 128, _E).transpose(0, 1, 3, 2) \
        .reshape(_N * _N * _E)
    tgt3d = target.astype(jnp.int32).reshape(_NT, 1, _BB)
    tx = target_x.astype(jnp.int32)

    nsteps = _NZ + _NT

    y, loss, ws = pl.pallas_call(
        _fused_body,
        grid_spec=pltpu.PrefetchScalarGridSpec(
            num_scalar_prefetch=1,
            grid=(nsteps,),
            in_specs=[
                pl.BlockSpec(memory_space=pl.ANY),
                pl.BlockSpec(memory_space=pl.ANY),
                pl.BlockSpec((1, _E), lambda g, tx: (0, 0)),
                pl.BlockSpec((_N, _DIN), lambda g, tx: (0, 0)),
                pl.BlockSpec((_DIN, _DOUT), lambda g, tx: (0, 0)),
                pl.BlockSpec((_DIN, _DOUT), lambda g, tx: (0, 0)),
                pl.BlockSpec((_DOUT, _DOUT), lambda g, tx: (0, 0)),
                pl.BlockSpec((2 * _DOUT, _DOUT), lambda g, tx: (0, 0)),
                pl.BlockSpec((1, _DOUT), lambda g, tx: (0, 0)),
                pl.BlockSpec((_DOUT, _NC), lambda g, tx: (0, 0)),
                pl.BlockSpec((1, _NC), lambda g, tx: (0, 0)),
                pl.BlockSpec((1, 1, _BB),
                             lambda g, tx: (jnp.maximum(g - _NZ, 0), 0, 0)),
            ],
            out_specs=[
                pl.BlockSpec((_BB, _NC),
                             lambda g, tx: (jnp.maximum(g - _NZ, 0), 0)),
                pl.BlockSpec((1, 1), lambda g, tx: (0, 0),
                             memory_space=pltpu.SMEM),
                pl.BlockSpec((1, _E), lambda g, tx: (0, 0)),
            ],
            scratch_shapes=[
                pltpu.VMEM((_N, _DOUT), f32),
                pltpu.VMEM((_N * _E, _DOUT), f32),
                pltpu.VMEM((_N, _DOUT), f32),
                pltpu.VMEM((_N, _DOUT), f32),
                pltpu.VMEM((2, _BMZ, _N), f32),
                pltpu.VMEM((2, _BB, _N * _E), f32),
                pltpu.VMEM((2, _BB, _N), f32),
                pltpu.SemaphoreType.DMA((2,)),
                pltpu.SemaphoreType.DMA((2,)),
                pltpu.SemaphoreType.DMA((2,)),
                pltpu.SMEM((1,), f32),
            ],
        ),
        out_shape=[
            jax.ShapeDtypeStruct((_B, _NC), f32),
            jax.ShapeDtypeStruct((1, 1), f32),
            jax.ShapeDtypeStruct((1, _E), f32),
        ],
        compiler_params=pltpu.CompilerParams(
            dimension_semantics=("arbitrary",)),
    )(tx, a_lin, H_adj, relw, X, W_sa, W1, W2, lin1_w,
      lin1_b.reshape(1, _DOUT), lin2_w, lin2_b.reshape(1, _NC), tgt3d)

    return (loss[0, 0], y, ws.reshape(1, _E, 1, 1))


# speedup vs baseline: 11.4100x; 1.0110x over previous
import jax
import jax.numpy as jnp
from jax import lax
from jax.experimental import pallas as pl
from jax.experimental.pallas import tpu as pltpu

_N = 4096
_E = 4
_DIN = 128
_DOUT = 64
_NC = 8
_B = 1024

_BB = 128
_NT = _B // _BB
_BMZ = 256
_NZ = _N // _BMZ


def _fused_body(tx_ref, a_ref, h_ref, relw_ref, x_ref, wsa_ref, w1_ref,
                w2_ref, l1w_ref, l1b_ref, l2w_ref, l2b_ref, tgt_ref,
                y_ref, loss_ref, ws_ref,
                u1_ref, vp_ref, zv1_ref, u2_ref, hslab, abuf, hbuf,
                zsem, asem, hsem, lacc):
    g = pl.program_id(0)

    def issue_tile(slot, tile):
        def ibody(i, _):
            r0 = tx_ref[tile * _BB + 2 * i]
            r1 = tx_ref[tile * _BB + 2 * i + 1]
            pltpu.make_async_copy(a_ref.at[pl.ds(r0 * (_N * _E), _N * _E)],
                                  abuf.at[slot, 2 * i],
                                  asem.at[slot, 0]).start()
            pltpu.make_async_copy(a_ref.at[pl.ds(r1 * (_N * _E), _N * _E)],
                                  abuf.at[slot, 2 * i + 1],
                                  asem.at[slot, 1]).start()
            pltpu.make_async_copy(h_ref.at[r0], hbuf.at[slot, 2 * i],
                                  hsem.at[slot]).start()
            pltpu.make_async_copy(h_ref.at[r1], hbuf.at[slot, 2 * i + 1],
                                  hsem.at[slot]).start()
            return 0
        lax.fori_loop(0, _BB // 2, ibody, 0, unroll=4)

    def issue_slab(slot, s):
        pltpu.make_async_copy(h_ref.at[pl.ds(s * _BMZ, _BMZ)],
                              hslab.at[slot], zsem.at[slot]).start()

    @pl.when(g == 0)
    def _():
        issue_slab(0, 0)
        issue_slab(1, 1)
        x = x_ref[:]
        u1_ref[:] = jnp.dot(x, w1_ref[:], preferred_element_type=jnp.float32)
        vsa = jnp.dot(x, wsa_ref[:], preferred_element_type=jnp.float32)
        rw = relw_ref[:]
        m = jnp.max(rw)
        ew = jnp.exp(rw - m)
        ws = ew / jnp.sum(ew)
        ws_ref[:] = ws
        for e in range(_E):
            sv = vsa * ws[0:1, e:e + 1]
            for jb in range(_N // 128):
                vp_ref[pl.ds(jb * 512 + e * 128, 128), :] = \
                    sv[jb * 128:(jb + 1) * 128, :]

    @pl.when(g == _NZ - 3)
    def _():
        issue_tile(0, 0)

    @pl.when(g < _NZ)
    def _():
        slot = g % 2
        pltpu.make_async_copy(hslab.at[slot], hslab.at[slot],
                              zsem.at[slot]).wait()
        zv1_ref[pl.ds(g * _BMZ, _BMZ), :] = jnp.maximum(
            jnp.dot(hslab[slot], u1_ref[:],
                    preferred_element_type=jnp.float32), 0.0)

        @pl.when(g + 2 < _NZ)
        def _():
            issue_slab(slot, g + 2)

    @pl.when(g >= _NZ)
    def _():
        t = g - _NZ

        @pl.when(t == 0)
        def _():
            u2_ref[:] = jnp.dot(zv1_ref[:], w2_ref[:],
                                preferred_element_type=jnp.float32)

        @pl.when(t + 1 < _NT)
        def _():
            issue_tile((t + 1) % 2, t + 1)

        slot = t % 2
        pltpu.make_async_copy(abuf.at[slot, pl.ds(0, _BB // 2)],
                              abuf.at[slot, pl.ds(0, _BB // 2)],
                              asem.at[slot, 0]).wait()
        pltpu.make_async_copy(abuf.at[slot, pl.ds(_BB // 2, _BB // 2)],
                              abuf.at[slot, pl.ds(_BB // 2, _BB // 2)],
                              asem.at[slot, 1]).wait()
        pltpu.make_async_copy(hbuf.at[slot], hbuf.at[slot],
                              hsem.at[slot]).wait()

        ag = abuf[slot]
        hg = hbuf[slot]
        zr = jnp.maximum(
            jnp.dot(ag, vp_ref[:], preferred_element_type=jnp.float32), 0.0)
        zv = jnp.dot(hg, u2_ref[:], preferred_element_type=jnp.float32)
        nrm = jnp.sqrt(jnp.sum(zr * zr, axis=1, keepdims=True) +
                       jnp.sum(zv * zv, axis=1, keepdims=True))
        inv = 1.0 / jnp.maximum(nrm, 1e-12)
        zrn = zr * inv
        zvn = zv * inv
        z1 = jnp.maximum(
            jnp.dot(zrn, l1w_ref[0:_DOUT, :],
                    preferred_element_type=jnp.float32)
            + jnp.dot(zvn, l1w_ref[_DOUT:, :],
                      preferred_element_type=jnp.float32)
            + l1b_ref[:], 0.0)
        yb = jnp.dot(z1, l2w_ref[:], preferred_element_type=jnp.float32) \
            + l2b_ref[:]
        y_ref[:] = yb
        m = jnp.max(yb, axis=1, keepdims=True)
        lse = m + jnp.log(jnp.sum(jnp.exp(yb - m), axis=1, keepdims=True))
        logp = yb - lse
        tt = tgt_ref[0, 0, :]
        sel = tt[:, None] == lax.broadcasted_iota(jnp.int32, (_BB, _NC), 1)
        contrib = jnp.sum(jnp.where(sel, logp, 0.0))
        prev = jnp.where(t == 0, 0.0, lacc[0])
        lacc[0] = prev + contrib

        @pl.when(t == _NT - 1)
        def _():
            loss_ref[0, 0] = -lacc[0] / _B


def kernel(A, H_adj, X, target_x, target, rel_weight, W_sa, W1, W2,
           lin1_w, lin1_b, lin2_w, lin2_b):
    f32 = jnp.float32
    relw = rel_weight.reshape(1, _E)

    a_lin = A.reshape(_N, _N // 128, 128, _E).transpose(0, 1, 3, 2) \
        .reshape(_N * _N * _E)
    tgt3d = target.astype(jnp.int32).reshape(_NT, 1, _BB)
    tx = target_x.astype(jnp.int32)

    nsteps = _NZ + _NT

    y, loss, ws = pl.pallas_call(
        _fused_body,
        grid_spec=pltpu.PrefetchScalarGridSpec(
            num_scalar_prefetch=1,
            grid=(nsteps,),
            in_specs=[
                pl.BlockSpec(memory_space=pl.ANY),
                pl.BlockSpec(memory_space=pl.ANY),
                pl.BlockSpec((1, _E), lambda g, tx: (0, 0)),
                pl.BlockSpec((_N, _DIN), lambda g, tx: (0, 0)),
                pl.BlockSpec((_DIN, _DOUT), lambda g, tx: (0, 0)),
                pl.BlockSpec((_DIN, _DOUT), lambda g, tx: (0, 0)),
                pl.BlockSpec((_DOUT, _DOUT), lambda g, tx: (0, 0)),
                pl.BlockSpec((2 * _DOUT, _DOUT), lambda g, tx: (0, 0)),
                pl.BlockSpec((1, _DOUT), lambda g, tx: (0, 0)),
                pl.BlockSpec((_DOUT, _NC), lambda g, tx: (0, 0)),
                pl.BlockSpec((1, _NC), lambda g, tx: (0, 0)),
                pl.BlockSpec((1, 1, _BB),
                             lambda g, tx: (jnp.maximum(g - _NZ, 0), 0, 0)),
            ],
            out_specs=[
                pl.BlockSpec((_BB, _NC),
                             lambda g, tx: (jnp.maximum(g - _NZ, 0), 0)),
                pl.BlockSpec((1, 1), lambda g, tx: (0, 0),
                             memory_space=pltpu.SMEM),
                pl.BlockSpec((1, _E), lambda g, tx: (0, 0)),
            ],
            scratch_shapes=[
                pltpu.VMEM((_N, _DOUT), f32),
                pltpu.VMEM((_N * _E, _DOUT), f32),
                pltpu.VMEM((_N, _DOUT), f32),
                pltpu.VMEM((_N, _DOUT), f32),
                pltpu.VMEM((2, _BMZ, _N), f32),
                pltpu.VMEM((2, _BB, _N * _E), f32),
                pltpu.VMEM((2, _BB, _N), f32),
                pltpu.SemaphoreType.DMA((2,)),
                pltpu.SemaphoreType.DMA((2, 2)),
                pltpu.SemaphoreType.DMA((2,)),
                pltpu.SMEM((1,), f32),
            ],
        ),
        out_shape=[
            jax.ShapeDtypeStruct((_B, _NC), f32),
            jax.ShapeDtypeStruct((1, 1), f32),
            jax.ShapeDtypeStruct((1, _E), f32),
        ],
        compiler_params=pltpu.CompilerParams(
            dimension_semantics=("arbitrary",)),
    )(tx, a_lin, H_adj, relw, X, W_sa, W1, W2, lin1_w,
      lin1_b.reshape(1, _DOUT), lin2_w, lin2_b.reshape(1, _NC), tgt3d)

    return (loss[0, 0], y, ws.reshape(1, _E, 1, 1))


# speedup vs baseline: 11.8510x; 1.0386x over previous
import jax
import jax.numpy as jnp
from jax import lax
from jax.experimental import pallas as pl
from jax.experimental.pallas import tpu as pltpu

_N = 4096
_E = 4
_DIN = 128
_DOUT = 64
_NC = 8
_B = 1024

_BB = 128
_NT = _B // _BB
_BMZ = 256
_NZ = _N // _BMZ


def _fused_body(tx_ref, a_ref, h_ref, relw_ref, x_ref, wsa_ref, w1_ref,
                w2_ref, l1w_ref, l1b_ref, l2w_ref, l2b_ref, tgt_ref,
                y_ref, loss_ref, ws_ref,
                u1_ref, vp_ref, zv1_ref, u2_ref, hslab, abuf, hbuf,
                zsem, asem, hsem, lacc):
    g = pl.program_id(0)

    def issue_tile(slot, tile):
        def ibody(i, _):
            r0 = tx_ref[tile * _BB + 2 * i]
            r1 = tx_ref[tile * _BB + 2 * i + 1]
            pltpu.make_async_copy(a_ref.at[pl.ds(r0 * (_N * _E), _N * _E)],
                                  abuf.at[slot, 2 * i],
                                  asem.at[slot, 0]).start()
            pltpu.make_async_copy(a_ref.at[pl.ds(r1 * (_N * _E), _N * _E)],
                                  abuf.at[slot, 2 * i + 1],
                                  asem.at[slot, 1]).start()
            pltpu.make_async_copy(h_ref.at[r0], hbuf.at[slot, 2 * i],
                                  hsem.at[slot]).start()
            pltpu.make_async_copy(h_ref.at[r1], hbuf.at[slot, 2 * i + 1],
                                  hsem.at[slot]).start()
            return 0
        lax.fori_loop(0, _BB // 2, ibody, 0, unroll=4)

    def issue_slab(slot, s):
        pltpu.make_async_copy(h_ref.at[pl.ds(s * _BMZ, _BMZ)],
                              hslab.at[slot], zsem.at[slot]).start()

    @pl.when(g == 0)
    def _():
        issue_slab(0, 0)
        issue_slab(1, 1)
        x = x_ref[:]
        u1_ref[:] = jnp.dot(x, w1_ref[:], preferred_element_type=jnp.float32)
        vsa = jnp.dot(x, wsa_ref[:], preferred_element_type=jnp.float32)
        rw = relw_ref[:]
        m = jnp.max(rw)
        ew = jnp.exp(rw - m)
        ws = ew / jnp.sum(ew)
        ws_ref[:] = ws
        for e in range(_E):
            sv = vsa * ws[0:1, e:e + 1]
            for jb in range(_N // 128):
                vp_ref[pl.ds(jb * 512 + e * 128, 128), :] = \
                    sv[jb * 128:(jb + 1) * 128, :]

    @pl.when(g == _NZ - 3)
    def _():
        issue_tile(0, 0)

    @pl.when(g < _NZ)
    def _():
        slot = g % 2
        pltpu.make_async_copy(hslab.at[slot], hslab.at[slot],
                              zsem.at[slot]).wait()
        zv1_ref[pl.ds(g * _BMZ, _BMZ), :] = jnp.maximum(
            jnp.dot(hslab[slot], u1_ref[:],
                    preferred_element_type=jnp.float32), 0.0)

        @pl.when(g + 2 < _NZ)
        def _():
            issue_slab(slot, g + 2)

    @pl.when(g >= _NZ)
    def _():
        t = g - _NZ

        @pl.when(t == 0)
        def _():
            u2_ref[:] = jnp.dot(zv1_ref[:], w2_ref[:],
                                preferred_element_type=jnp.float32)

        @pl.when(t + 1 < _NT)
        def _():
            issue_tile((t + 1) % 2, t + 1)

        slot = t % 2
        pltpu.make_async_copy(abuf.at[slot, pl.ds(0, _BB // 2)],
                              abuf.at[slot, pl.ds(0, _BB // 2)],
                              asem.at[slot, 0]).wait()
        pltpu.make_async_copy(abuf.at[slot, pl.ds(_BB // 2, _BB // 2)],
                              abuf.at[slot, pl.ds(_BB // 2, _BB // 2)],
                              asem.at[slot, 1]).wait()
        pltpu.make_async_copy(hbuf.at[slot], hbuf.at[slot],
                              hsem.at[slot]).wait()

        ag = abuf[slot]
        hg = hbuf[slot]
        zr = jnp.maximum(
            jnp.dot(ag, vp_ref[:], preferred_element_type=jnp.float32), 0.0)
        zv = jnp.dot(hg, u2_ref[:], preferred_element_type=jnp.float32)
        nrm = jnp.sqrt(jnp.sum(zr * zr, axis=1, keepdims=True) +
                       jnp.sum(zv * zv, axis=1, keepdims=True))
        inv = 1.0 / jnp.maximum(nrm, 1e-12)
        zrn = zr * inv
        zvn = zv * inv
        dn = (((1,), (1,)), ((), ()))
        z1 = jnp.maximum(
            lax.dot_general(zrn, l1w_ref[:, 0:_DOUT],
                            dn, preferred_element_type=jnp.float32)
            + lax.dot_general(zvn, l1w_ref[:, _DOUT:],
                              dn, preferred_element_type=jnp.float32)
            + l1b_ref[:], 0.0)
        yb = lax.dot_general(z1, l2w_ref[:], dn,
                             preferred_element_type=jnp.float32) \
            + l2b_ref[:]
        y_ref[:] = yb
        m = jnp.max(yb, axis=1, keepdims=True)
        lse = m + jnp.log(jnp.sum(jnp.exp(yb - m), axis=1, keepdims=True))
        logp = yb - lse
        tt = tgt_ref[0, 0, :]
        sel = tt[:, None] == lax.broadcasted_iota(jnp.int32, (_BB, _NC), 1)
        contrib = jnp.sum(jnp.where(sel, logp, 0.0))
        prev = jnp.where(t == 0, 0.0, lacc[0])
        lacc[0] = prev + contrib

        @pl.when(t == _NT - 1)
        def _():
            loss_ref[0, 0] = -lacc[0] / _B


def kernel(A, H_adj, X, target_x, target, rel_weight, W_sa, W1, W2,
           lin1_w, lin1_b, lin2_w, lin2_b):
    f32 = jnp.float32
    relw = rel_weight.reshape(1, _E)

    a_lin = A.reshape(_N, _N // 128, 128, _E).transpose(0, 1, 3, 2) \
        .reshape(_N * _N * _E)
    tgt3d = target.astype(jnp.int32).reshape(_NT, 1, _BB)
    tx = target_x.astype(jnp.int32)

    nsteps = _NZ + _NT

    y, loss, ws = pl.pallas_call(
        _fused_body,
        grid_spec=pltpu.PrefetchScalarGridSpec(
            num_scalar_prefetch=1,
            grid=(nsteps,),
            in_specs=[
                pl.BlockSpec(memory_space=pl.ANY),
                pl.BlockSpec(memory_space=pl.ANY),
                pl.BlockSpec((1, _E), lambda g, tx: (0, 0)),
                pl.BlockSpec((_N, _DIN), lambda g, tx: (0, 0)),
                pl.BlockSpec((_DIN, _DOUT), lambda g, tx: (0, 0)),
                pl.BlockSpec((_DIN, _DOUT), lambda g, tx: (0, 0)),
                pl.BlockSpec((_DOUT, _DOUT), lambda g, tx: (0, 0)),
                pl.BlockSpec((_DOUT, 2 * _DOUT), lambda g, tx: (0, 0)),
                pl.BlockSpec((1, _DOUT), lambda g, tx: (0, 0)),
                pl.BlockSpec((_NC, _DOUT), lambda g, tx: (0, 0)),
                pl.BlockSpec((1, _NC), lambda g, tx: (0, 0)),
                pl.BlockSpec((1, 1, _BB),
                             lambda g, tx: (jnp.maximum(g - _NZ, 0), 0, 0)),
            ],
            out_specs=[
                pl.BlockSpec((_BB, _NC),
                             lambda g, tx: (jnp.maximum(g - _NZ, 0), 0)),
                pl.BlockSpec((1, 1), lambda g, tx: (0, 0),
                             memory_space=pltpu.SMEM),
                pl.BlockSpec((1, _E), lambda g, tx: (0, 0)),
            ],
            scratch_shapes=[
                pltpu.VMEM((_N, _DOUT), f32),
                pltpu.VMEM((_N * _E, _DOUT), f32),
                pltpu.VMEM((_N, _DOUT), f32),
                pltpu.VMEM((_N, _DOUT), f32),
                pltpu.VMEM((2, _BMZ, _N), f32),
                pltpu.VMEM((2, _BB, _N * _E), f32),
                pltpu.VMEM((2, _BB, _N), f32),
                pltpu.SemaphoreType.DMA((2,)),
                pltpu.SemaphoreType.DMA((2, 2)),
                pltpu.SemaphoreType.DMA((2,)),
                pltpu.SMEM((1,), f32),
            ],
        ),
        out_shape=[
            jax.ShapeDtypeStruct((_B, _NC), f32),
            jax.ShapeDtypeStruct((1, 1), f32),
            jax.ShapeDtypeStruct((1, _E), f32),
        ],
        compiler_params=pltpu.CompilerParams(
            dimension_semantics=("arbitrary",)),
    )(tx, a_lin, H_adj, relw, X, W_sa, W1, W2, lin1_w.T,
      lin1_b.reshape(1, _DOUT), lin2_w.T, lin2_b.reshape(1, _NC), tgt3d)

    return (loss[0, 0], y, ws.reshape(1, _E, 1, 1))


# speedup vs baseline: 12.1648x; 1.0265x over previous
import jax
import jax.numpy as jnp
from jax import lax
from jax.experimental import pallas as pl
from jax.experimental.pallas import tpu as pltpu

_N = 4096
_E = 4
_DIN = 128
_DOUT = 64
_NC = 8
_B = 1024

_BB = 128
_NT = _B // _BB
_BMZ = 256
_NZ = _N // _BMZ


def _fused_body(tx_ref, a_ref, h_ref, relw_ref, x_ref, wsa_ref, w1_ref,
                w2_ref, l1w_ref, l1b_ref, l2w_ref, l2b_ref, tgt_ref,
                y_ref, loss_ref, ws_ref,
                u1_ref, vp_ref, zv1_ref, u2_ref, hslab, abuf, hbuf,
                zsem, asem, hsem, lacc):
    g = pl.program_id(0)

    def issue_tile(slot, tile):
        def ibody(i, _):
            r0 = tx_ref[tile * _BB + 2 * i]
            r1 = tx_ref[tile * _BB + 2 * i + 1]
            pltpu.make_async_copy(a_ref.at[pl.ds(r0 * (_N * _E), _N * _E)],
                                  abuf.at[slot, 2 * i],
                                  asem.at[slot, 0]).start()
            pltpu.make_async_copy(a_ref.at[pl.ds(r1 * (_N * _E), _N * _E)],
                                  abuf.at[slot, 2 * i + 1],
                                  asem.at[slot, 1]).start()
            pltpu.make_async_copy(h_ref.at[r0], hbuf.at[slot, 2 * i],
                                  hsem.at[slot]).start()
            pltpu.make_async_copy(h_ref.at[r1], hbuf.at[slot, 2 * i + 1],
                                  hsem.at[slot]).start()
            return 0
        lax.fori_loop(0, _BB // 2, ibody, 0, unroll=4)

    def issue_slab(slot, s):
        pltpu.make_async_copy(h_ref.at[pl.ds(s * _BMZ, _BMZ)],
                              hslab.at[slot], zsem.at[slot]).start()

    @pl.when(g == 0)
    def _():
        issue_slab(0, 0)
        issue_slab(1, 1)
        x = x_ref[:]
        u1_ref[:] = jnp.dot(x, w1_ref[:], preferred_element_type=jnp.float32)
        vsa = jnp.dot(x, wsa_ref[:], preferred_element_type=jnp.float32)
        rw = relw_ref[:]
        m = jnp.max(rw)
        ew = jnp.exp(rw - m)
        ws = ew / jnp.sum(ew)
        ws_ref[:] = ws
        for e in range(_E):
            sv = vsa * ws[0:1, e:e + 1]
            for jb in range(_N // 128):
                vp_ref[pl.ds(jb * 512 + e * 128, 128), :] = \
                    sv[jb * 128:(jb + 1) * 128, :]

    @pl.when(g == _NZ - 3)
    def _():
        issue_tile(0, 0)

    @pl.when(g == _NZ - 2)
    def _():
        issue_tile(1, 1)

    @pl.when(g < _NZ)
    def _():
        slot = g % 2
        pltpu.make_async_copy(hslab.at[slot], hslab.at[slot],
                              zsem.at[slot]).wait()
        zv1_ref[pl.ds(g * _BMZ, _BMZ), :] = jnp.maximum(
            jnp.dot(hslab[slot], u1_ref[:],
                    preferred_element_type=jnp.float32), 0.0)

        @pl.when(g + 2 < _NZ)
        def _():
            issue_slab(slot, g + 2)

    @pl.when(g >= _NZ)
    def _():
        t = g - _NZ

        @pl.when(t == 0)
        def _():
            u2_ref[:] = jnp.dot(zv1_ref[:], w2_ref[:],
                                preferred_element_type=jnp.float32)

        @pl.when(t + 2 < _NT)
        def _():
            issue_tile((t + 2) % 3, t + 2)

        slot = t % 3
        pltpu.make_async_copy(abuf.at[slot, pl.ds(0, _BB // 2)],
                              abuf.at[slot, pl.ds(0, _BB // 2)],
                              asem.at[slot, 0]).wait()
        pltpu.make_async_copy(abuf.at[slot, pl.ds(_BB // 2, _BB // 2)],
                              abuf.at[slot, pl.ds(_BB // 2, _BB // 2)],
                              asem.at[slot, 1]).wait()
        pltpu.make_async_copy(hbuf.at[slot], hbuf.at[slot],
                              hsem.at[slot]).wait()

        ag = abuf[slot]
        hg = hbuf[slot]
        zr = jnp.maximum(
            jnp.dot(ag, vp_ref[:], preferred_element_type=jnp.float32), 0.0)
        zv = jnp.dot(hg, u2_ref[:], preferred_element_type=jnp.float32)
        nrm = jnp.sqrt(jnp.sum(zr * zr, axis=1, keepdims=True) +
                       jnp.sum(zv * zv, axis=1, keepdims=True))
        inv = 1.0 / jnp.maximum(nrm, 1e-12)
        zrn = zr * inv
        zvn = zv * inv
        dn = (((1,), (1,)), ((), ()))
        z1 = jnp.maximum(
            lax.dot_general(zrn, l1w_ref[:, 0:_DOUT],
                            dn, preferred_element_type=jnp.float32)
            + lax.dot_general(zvn, l1w_ref[:, _DOUT:],
                              dn, preferred_element_type=jnp.float32)
            + l1b_ref[:], 0.0)
        yb = lax.dot_general(z1, l2w_ref[:], dn,
                             preferred_element_type=jnp.float32) \
            + l2b_ref[:]
        y_ref[:] = yb
        m = jnp.max(yb, axis=1, keepdims=True)
        lse = m + jnp.log(jnp.sum(jnp.exp(yb - m), axis=1, keepdims=True))
        logp = yb - lse
        tt = tgt_ref[0, 0, :]
        sel = tt[:, None] == lax.broadcasted_iota(jnp.int32, (_BB, _NC), 1)
        contrib = jnp.sum(jnp.where(sel, logp, 0.0))
        prev = jnp.where(t == 0, 0.0, lacc[0])
        lacc[0] = prev + contrib

        @pl.when(t == _NT - 1)
        def _():
            loss_ref[0, 0] = -lacc[0] / _B


def kernel(A, H_adj, X, target_x, target, rel_weight, W_sa, W1, W2,
           lin1_w, lin1_b, lin2_w, lin2_b):
    f32 = jnp.float32
    relw = rel_weight.reshape(1, _E)

    a_lin = A.reshape(_N, _N // 128, 128, _E).transpose(0, 1, 3, 2) \
        .reshape(_N * _N * _E)
    tgt3d = target.astype(jnp.int32).reshape(_NT, 1, _BB)
    tx = target_x.astype(jnp.int32)

    nsteps = _NZ + _NT

    y, loss, ws = pl.pallas_call(
        _fused_body,
        grid_spec=pltpu.PrefetchScalarGridSpec(
            num_scalar_prefetch=1,
            grid=(nsteps,),
            in_specs=[
                pl.BlockSpec(memory_space=pl.ANY),
                pl.BlockSpec(memory_space=pl.ANY),
                pl.BlockSpec((1, _E), lambda g, tx: (0, 0)),
                pl.BlockSpec((_N, _DIN), lambda g, tx: (0, 0)),
                pl.BlockSpec((_DIN, _DOUT), lambda g, tx: (0, 0)),
                pl.BlockSpec((_DIN, _DOUT), lambda g, tx: (0, 0)),
                pl.BlockSpec((_DOUT, _DOUT), lambda g, tx: (0, 0)),
                pl.BlockSpec((_DOUT, 2 * _DOUT), lambda g, tx: (0, 0)),
                pl.BlockSpec((1, _DOUT), lambda g, tx: (0, 0)),
                pl.BlockSpec((_NC, _DOUT), lambda g, tx: (0, 0)),
                pl.BlockSpec((1, _NC), lambda g, tx: (0, 0)),
                pl.BlockSpec((1, 1, _BB),
                             lambda g, tx: (jnp.maximum(g - _NZ, 0), 0, 0)),
            ],
            out_specs=[
                pl.BlockSpec((_BB, _NC),
                             lambda g, tx: (jnp.maximum(g - _NZ, 0), 0)),
                pl.BlockSpec((1, 1), lambda g, tx: (0, 0),
                             memory_space=pltpu.SMEM),
                pl.BlockSpec((1, _E), lambda g, tx: (0, 0)),
            ],
            scratch_shapes=[
                pltpu.VMEM((_N, _DOUT), f32),
                pltpu.VMEM((_N * _E, _DOUT), f32),
                pltpu.VMEM((_N, _DOUT), f32),
                pltpu.VMEM((_N, _DOUT), f32),
                pltpu.VMEM((2, _BMZ, _N), f32),
                pltpu.VMEM((3, _BB, _N * _E), f32),
                pltpu.VMEM((3, _BB, _N), f32),
                pltpu.SemaphoreType.DMA((2,)),
                pltpu.SemaphoreType.DMA((3, 2)),
                pltpu.SemaphoreType.DMA((3,)),
                pltpu.SMEM((1,), f32),
            ],
        ),
        out_shape=[
            jax.ShapeDtypeStruct((_B, _NC), f32),
            jax.ShapeDtypeStruct((1, 1), f32),
            jax.ShapeDtypeStruct((1, _E), f32),
        ],
        compiler_params=pltpu.CompilerParams(
            dimension_semantics=("arbitrary",)),
    )(tx, a_lin, H_adj, relw, X, W_sa, W1, W2, lin1_w.T,
      lin1_b.reshape(1, _DOUT), lin2_w.T, lin2_b.reshape(1, _NC), tgt3d)

    return (loss[0, 0], y, ws.reshape(1, _E, 1, 1))
